# baseline (speedup 1.0000x reference)
import functools

import jax
import jax.numpy as jnp
import numpy as np
from jax.experimental import pallas as pl
from jax.experimental.pallas import tpu as pltpu

_OMEGA = 3.0
_EPSILON = 2.0
_WING_C = 3.0 - 3.0 * float(np.log1p(_OMEGA / _EPSILON))

_INT_MIN = np.int32(-(2 ** 31))
_INT_MAX = np.int32(2 ** 31 - 1)
_XOR = np.int32(0x7FFFFFFF)

_RB = 32
_BL = 256


def _keys_from(vals):
    bits = jax.lax.bitcast_convert_type(vals, jnp.int32)
    return jnp.where(bits < 0, jnp.bitwise_xor(bits, _XOR), bits)


def _vals_from(keys):
    return jax.lax.bitcast_convert_type(
        jnp.where(keys < 0, jnp.bitwise_xor(keys, _XOR), keys), jnp.float32)


def _body(annb_ref, ann_ref, anc_pk_ref, anc_pad_ref, cls_ref, bbox_ref,
          ldm_ref, out_ref, keys_scr, cnt_scr, acc, ldm_buf, sem,
          *, nba, n_ann, batch, a_real, nbl):
    j = pl.program_id(0)
    i = pl.program_id(1)
    rb = _RB
    bap = rb * 128

    @pl.when(jnp.logical_and(j == 0, i == 0))
    def _():
        acc[6] = 0.0
        acc[7] = 0.0
        acc[8] = 0.0

    @pl.when(i == 0)
    def _():
        acc[0] = 0.0
        acc[1] = 0.0
        acc[2] = 0.0
        acc[3] = 0.0
        acc[4] = 0.0
        acc[5] = 0.0

    annb = annb_ref[0]
    b0 = annb[0][:, :, None]
    b1 = annb[1][:, :, None]
    b2 = annb[2][:, :, None]
    b3 = annb[3][:, :, None]
    valid3 = b0 > 0.0

    anc = anc_pk_ref[...]
    a0 = anc[0]
    a1 = anc[1]
    a2 = anc[2]
    a3 = anc[3]

    iw = jnp.maximum(jnp.minimum(a2[None], b2) - jnp.maximum(a0[None], b0),
                     0.0)
    ih = jnp.maximum(jnp.minimum(a3[None], b3) - jnp.maximum(a1[None], b1),
                     0.0)
    inter = iw * ih
    ua = jnp.maximum((a2 - a0)[None] * (a3 - a1)[None]
                     + (b2 - b0) * (b3 - b1) - inter, 1e-08)
    iou = jnp.where(valid3, inter / ua, -1.0)
    iou_max = jnp.max(iou, axis=0)
    rows3 = jax.lax.broadcasted_iota(jnp.int32, iou.shape, 0)
    idx = jnp.min(jnp.where(iou == iou_max[None], rows3, n_ann), axis=0)
    eq3 = rows3 == idx[None]

    gidx = (i * bap
            + jax.lax.broadcasted_iota(jnp.int32, (rb, 128), 0) * 128
            + jax.lax.broadcasted_iota(jnp.int32, (rb, 128), 1))
    inb = gidx < a_real
    pos = jnp.logical_and(iou_max >= 0.7, inb)
    neg = jnp.logical_and(iou_max < 0.4, inb)
    posf = jnp.where(pos, 1.0, 0.0)
    acc[0] = acc[0] + jnp.sum(posf)
    acc[1] = acc[1] + jnp.sum(jnp.where(neg, 1.0, 0.0))

    cls = cls_ref[0]
    acc[2] = acc[2] + jnp.sum(jnp.where(pos, -cls[0], 0.0))
    neg_vals = jnp.where(neg, -cls[1], -jnp.inf)
    keys_scr[pl.ds(i, 1)] = _keys_from(neg_vals)[None]

    spb = _BL // 128
    for k in range(rb // spb):
        cnt_scr[i * (rb // spb) + k] = jnp.sum(
            jnp.where(pos[k * spb:(k + 1) * spb, :], 1, 0))

    g0 = jnp.sum(jnp.where(eq3, b0, 0.0), axis=0)
    g1 = jnp.sum(jnp.where(eq3, b1, 0.0), axis=0)
    g2 = jnp.sum(jnp.where(eq3, b2, 0.0), axis=0)
    g3 = jnp.sum(jnp.where(eq3, b3, 0.0), axis=0)
    aw = a2 - a0
    ah = a3 - a1
    acx = a0 + 0.5 * aw
    acy = a1 + 0.5 * ah
    gw = g2 - g0
    gh = g3 - g1
    gcx = g0 + 0.5 * gw
    gcy = g1 + 0.5 * gh
    bb = bbox_ref[0]
    tdx = (gcx - acx) / (aw + 1e-14) / 0.1
    tdy = (gcy - acy) / (ah + 1e-14) / 0.1
    tdw = jnp.log(jnp.where(pos, gw, aw) / aw) / 0.2
    tdh = jnp.log(jnp.where(pos, gh, ah) / ah) / 0.2
    s_acc = 0.0
    for td, breg in ((tdx, bb[0]), (tdy, bb[1]), (tdw, bb[2]), (tdh, bb[3])):
        diff = jnp.abs(td - breg)
        sl1 = jnp.where(diff < 1.0, 0.5 * diff * diff, diff - 0.5)
        s_acc = s_acc + jnp.sum(jnp.where(pos, sl1, 0.0))
    acc[3] = acc[3] + s_acc

    @pl.when(i == nba - 1)
    def _():
        num_pos = acc[0]
        num_neg = acc[1]
        count_f = jnp.minimum(num_pos * 3.0, num_neg)
        count = count_f.astype(jnp.int32)
        keys_all = keys_scr[...]

        def step(_, lohi):
            lo, hi = lohi
            mid = (lo >> 1) + (hi >> 1) + ((lo | hi) & 1)
            c_ge = jnp.sum(jnp.where(keys_all >= mid, 1, 0))
            ok = c_ge >= count
            lo2 = jnp.where(ok, mid, lo)
            hi2 = jnp.where(ok, hi, mid - 1)
            cont = lo < hi
            return (jnp.where(cont, lo2, lo), jnp.where(cont, hi2, hi))

        kth, _ = jax.lax.fori_loop(0, 33, step, (_INT_MIN, _INT_MAX))
        vals_all = _vals_from(keys_all)
        gt = keys_all > kth
        c_gt = jnp.sum(jnp.where(gt, 1.0, 0.0))
        sum_gt = jnp.sum(jnp.where(gt, vals_all, 0.0))
        val_k = _vals_from(kth)
        neg_sum = sum_gt + (count_f - c_gt) * val_k
        neg_mean = jnp.where(count_f > 0.0,
                             neg_sum / jnp.maximum(count_f, 1.0), 0.0)

        ann = ann_ref[0]
        ldm_ann = ann[:, 4:]
        s_col = jnp.sum(ldm_ann, axis=1, keepdims=True)
        rhs = jnp.concatenate([ldm_ann, s_col], axis=1)

        def ldm_block(g, carry):
            wl_s, nl_s = carry

            def active():
                base = jnp.minimum(g * _BL, a_real - _BL)
                cp = pltpu.make_async_copy(
                    ldm_ref.at[j, pl.ds(base, _BL), :], ldm_buf, sem)
                cp.start()
                anc_b = anc_pad_ref[pl.ds(base, _BL), :]
                a0c = anc_b[:, 0:1]
                a1c = anc_b[:, 1:2]
                a2c = anc_b[:, 2:3]
                a3c = anc_b[:, 3:4]
                br0 = annb[0][None, :, 0]
                br1 = annb[1][None, :, 0]
                br2 = annb[2][None, :, 0]
                br3 = annb[3][None, :, 0]
                vrow = br0 > 0.0
                iwc = jnp.maximum(jnp.minimum(a2c, br2)
                                  - jnp.maximum(a0c, br0), 0.0)
                ihc = jnp.maximum(jnp.minimum(a3c, br3)
                                  - jnp.maximum(a1c, br1), 0.0)
                intc = iwc * ihc
                uac = jnp.maximum((a2c - a0c) * (a3c - a1c)
                                  + (br2 - br0) * (br3 - br1) - intc, 1e-08)
                iouc = jnp.where(vrow, intc / uac, -1.0)
                mxc = jnp.max(iouc, axis=1, keepdims=True)
                colc = jax.lax.broadcasted_iota(jnp.int32, iouc.shape, 1)
                idxc = jnp.min(jnp.where(iouc == mxc, colc, n_ann),
                               axis=1, keepdims=True)
                ohc = (colc == idxc).astype(jnp.float32)
                lidx = (base
                        + jax.lax.broadcasted_iota(jnp.int32, (_BL, 1), 0))
                own = jnp.logical_and(lidx >= g * _BL, lidx < a_real)
                posc = jnp.logical_and(mxc >= 0.7, own)
                asg = jnp.dot(ohc, rhs, preferred_element_type=jnp.float32)
                assigned = asg[:, :196]
                rs = asg[:, 196:197]
                lpos = jnp.logical_and(rs > 0.0, posc)
                awc = a2c - a0c
                ahc = a3c - a1c
                acxc = a0c + 0.5 * awc
                acyc = a1c + 0.5 * ahc
                isx = (jax.lax.broadcasted_iota(jnp.int32, (1, 196), 1)
                       % 2) == 0
                denom = jnp.where(isx, awc, ahc) + 1e-14
                ctr = jnp.where(isx, acxc, acyc)
                cp.wait()
                lt = (assigned - ctr) / denom / 0.1
                delta = jnp.abs(lt - ldm_buf[...])
                wl = jnp.where(delta < _OMEGA,
                               _OMEGA * jnp.log1p(delta / _EPSILON),
                               delta - _WING_C)
                return (wl_s + jnp.sum(jnp.where(lpos, wl, 0.0)),
                        nl_s + jnp.sum(jnp.where(lpos, 1.0, 0.0)))

            has_pos = cnt_scr[g] > 0
            return jax.lax.cond(has_pos, active, lambda: (wl_s, nl_s))

        wl_sum, num_lpos = jax.lax.fori_loop(0, nbl, ldm_block, (0.0, 0.0))
        acc[4] = wl_sum
        acc[5] = num_lpos

        has_ann = jnp.max(jnp.where(valid3[:, 0, 0], 1.0, 0.0))
        pos_mean = acc[2] / jnp.maximum(num_pos, 1.0)
        cls_l = jnp.where(num_pos > 0.0, pos_mean + neg_mean, 0.0) * has_ann
        box_l = jnp.where(num_pos > 0.0,
                          acc[3] / jnp.maximum(num_pos * 4.0, 1.0),
                          0.0) * has_ann
        ldm_l = jnp.where(acc[5] > 0.0,
                          acc[4] / jnp.maximum(acc[5] * 196.0, 1.0),
                          0.0) * has_ann
        acc[6] = acc[6] + cls_l / batch
        acc[7] = acc[7] + box_l / batch
        acc[8] = acc[8] + ldm_l / batch
        out_ref[:, :] = jnp.concatenate(
            [jnp.broadcast_to(acc[6], (1, 1)),
             jnp.broadcast_to(acc[7], (1, 1)),
             jnp.broadcast_to(acc[8], (1, 1))], axis=1)


def kernel(classifications, bbox_regressions, ldm_regressions, anchors,
           annotations):
    B, A, _ = classifications.shape
    N = annotations.shape[1]
    bap = _RB * 128
    nba = -(-A // bap)
    ap = nba * bap
    nbl = -(-A // _BL)
    pad = ap - A

    anc = anchors[0]
    anc_pad = jnp.pad(anc, ((0, pad), (0, 0)))
    anc_pk = anc_pad.T.reshape(4, ap // 128, 128)
    cls_pk = jnp.pad(classifications, ((0, 0), (0, pad), (0, 0))) \
        .transpose(0, 2, 1).reshape(B, 2, ap // 128, 128)
    bbox_pk = jnp.pad(bbox_regressions, ((0, 0), (0, pad), (0, 0))) \
        .transpose(0, 2, 1).reshape(B, 4, ap // 128, 128)
    annb = annotations[:, :, :4].transpose(0, 2, 1)[..., None]

    body = functools.partial(_body, nba=nba, n_ann=N, batch=float(B),
                             a_real=A, nbl=nbl)
    out = pl.pallas_call(
        body,
        grid=(B, nba),
        in_specs=[
            pl.BlockSpec((1, 4, N, 1), lambda j, i: (j, 0, 0, 0)),
            pl.BlockSpec((1, N, 200), lambda j, i: (j, 0, 0)),
            pl.BlockSpec((4, _RB, 128), lambda j, i: (0, i, 0)),
            pl.BlockSpec((ap, 4), lambda j, i: (0, 0)),
            pl.BlockSpec((1, 2, _RB, 128), lambda j, i: (j, 0, i, 0)),
            pl.BlockSpec((1, 4, _RB, 128), lambda j, i: (j, 0, i, 0)),
            pl.BlockSpec(memory_space=pl.ANY),
        ],
        out_specs=pl.BlockSpec((1, 3), lambda j, i: (0, 0)),
        out_shape=jax.ShapeDtypeStruct((1, 3), jnp.float32),
        scratch_shapes=[
            pltpu.VMEM((nba, _RB, 128), jnp.int32),
            pltpu.SMEM((256,), jnp.int32),
            pltpu.SMEM((16,), jnp.float32),
            pltpu.VMEM((_BL, 196), jnp.float32),
            pltpu.SemaphoreType.DMA,
        ],
        compiler_params=pltpu.CompilerParams(
            dimension_semantics=("arbitrary", "arbitrary")),
    )(annb, annotations, anc_pk, anc_pad, cls_pk, bbox_pk, ldm_regressions)
    return out[0]
